# baseline (speedup 1.0000x reference)
import functools

import jax
import jax.numpy as jnp
from jax import lax
from jax.experimental import pallas as pl
from jax.experimental.pallas import tpu as pltpu
from jax.experimental.pallas import tpu_sc as plsc

NC = 2
NS = 16
L = 16
NW = NC * NS
W = 128


def _pack_table(table, D):
    V = table.shape[0]
    tb = table.astype(jnp.bfloat16)
    packed = jax.lax.bitcast_convert_type(
        tb.reshape(V, D // 2, 2), jnp.uint32)
    return packed.reshape(V // 4, W)


def _make_sc_kernel(B, K, D, V):
    BW = B // NW
    C = 16
    NCH = BW // C
    Q = D // L
    SU = D // 2
    CK = C * K
    NSTR = -(-CK // 128)

    mesh = plsc.VectorSubcoreMesh(core_axis_name="c", subcore_axis_name="s")

    @functools.partial(
        pl.kernel,
        out_type=jax.ShapeDtypeStruct((B * 2 * L,), jnp.float32),
        mesh=mesh,
        scratch_types=[
            pltpu.VMEM((BW,), jnp.int32),
            pltpu.VMEM((BW,), jnp.int32),
            pltpu.VMEM((BW * K,), jnp.int32),
            pltpu.VMEM((C,), jnp.int32),
            pltpu.VMEM((C,), jnp.int32),
            pltpu.VMEM((CK + 4 * L,), jnp.int32),
            pltpu.VMEM((C, W), jnp.uint32),
            pltpu.VMEM((C, W), jnp.uint32),
            pltpu.VMEM((CK, W), jnp.uint32),
            pltpu.VMEM((C * 2 * L,), jnp.float32),
            pltpu.SemaphoreType.DMA,
        ],
        compiler_params=pltpu.CompilerParams(needs_layout_passes=False),
    )
    def sg_kernel(tw_hbm, cw_hbm, neg_hbm, tt_hbm, ct_hbm, out_hbm,
                  idx_t, idx_c, idx_n, qid_t, qid_c, qid_n,
                  rows_t, rows_c, rows_n, acc, sem):
        wid = lax.axis_index("s") * NC + lax.axis_index("c")
        base_w = wid * BW

        pltpu.sync_copy(tw_hbm.at[pl.ds(base_w, BW)], idx_t)
        pltpu.sync_copy(cw_hbm.at[pl.ds(base_w, BW)], idx_c)
        pltpu.sync_copy(neg_hbm.at[pl.ds(base_w * K, BW * K)], idx_n)

        lane = lax.iota(jnp.int32, L)

        def chunk_body(ci, carry):
            base = ci * C
            tvec = idx_t[pl.ds(base, L)]
            cvec = idx_c[pl.ds(base, L)]
            qid_t[pl.ds(0, L)] = tvec >> 2
            qid_c[pl.ds(0, L)] = cvec >> 2
            nvecs = []
            for i in range(CK // L):
                nv = idx_n[pl.ds(base * K + i * L, L)]
                qid_n[pl.ds(i * L, L)] = nv >> 2
                nvecs.append(nv)

            copies = [
                pltpu.async_copy(tt_hbm.at[qid_t], rows_t, sem),
                pltpu.async_copy(ct_hbm.at[qid_c], rows_c, sem),
            ]
            for j in range(NSTR):
                n = min(128, CK - j * 128)
                copies.append(
                    pltpu.async_copy(ct_hbm.at[qid_n.at[pl.ds(j * 128, n)]],
                                     rows_n.at[pl.ds(j * 128, n)], sem))
            for cp in copies:
                cp.wait()

            def row_f32(ref, r, sub):
                out = []
                for h in range(SU // L):
                    w = ref[r, pl.ds(sub + h * L, L)]
                    ab = plsc.bitcast(w, jnp.bfloat16)
                    a, b = plsc.unpack(ab, format=plsc.PackFormat.INTERLEAVED,
                                       preferred_element_type=jnp.float32)
                    out += [a, b]
                return out

            for e in range(C):
                t = row_f32(rows_t, e, (tvec[e] & 3) * SU)
                c = row_f32(rows_c, e, (cvec[e] & 3) * SU)
                p = t[0] * c[0]
                for q in range(1, Q):
                    p = p + t[q] * c[q]
                v0 = jnp.where(lane == 0, jnp.sum(p), 0.0)
                v1 = jnp.zeros((L,), jnp.float32)
                for k in range(K):
                    j = e * K + k
                    nv = nvecs[j // L]
                    n = row_f32(rows_n, j, (nv[j % L] & 3) * SU)
                    s = t[0] * n[0]
                    for q in range(1, Q):
                        s = s + t[q] * n[q]
                    col = 1 + k
                    if col < L:
                        v0 = jnp.where(lane == col, jnp.sum(s), v0)
                    else:
                        v1 = jnp.where(lane == col - L, jnp.sum(s), v1)
                acc[pl.ds(e * 2 * L, L)] = v0
                acc[pl.ds(e * 2 * L + L, L)] = v1

            pltpu.sync_copy(
                acc, out_hbm.at[pl.ds((base_w + base) * 2 * L, C * 2 * L)])
            return carry

        lax.fori_loop(0, NCH, chunk_body, 0)

    return sg_kernel


def kernel(target_word, context_word, negative_samples, target_table, context_table):
    B = target_word.shape[0]
    K = negative_samples.shape[1]
    V, D = target_table.shape
    tw = target_word.astype(jnp.int32)
    cw = context_word.astype(jnp.int32)
    neg = negative_samples.astype(jnp.int32).reshape(B * K)
    ttq = _pack_table(target_table, D)
    ctq = _pack_table(context_table, D)
    sg = _make_sc_kernel(B, K, D, V)
    out = sg(tw, cw, neg, ttq, ctq)
    return out.reshape(B, 2 * L)[:, :1 + K]


# speedup vs baseline: 3.4723x; 3.4723x over previous
import functools

import jax
import jax.numpy as jnp
from jax import lax
from jax.experimental import pallas as pl
from jax.experimental.pallas import tpu as pltpu
from jax.experimental.pallas import tpu_sc as plsc

NC = 2
NS = 16
L = 16
NW = NC * NS
W = 128


def _make_sc_kernel(B, K, D, V):
    BW = B // NW
    C = 16
    NCH = BW // C
    Q = D // L
    CK = C * K
    NSTR = -(-CK // 128)

    mesh = plsc.VectorSubcoreMesh(core_axis_name="c", subcore_axis_name="s")

    @functools.partial(
        pl.kernel,
        out_type=jax.ShapeDtypeStruct((B * 2 * L,), jnp.float32),
        mesh=mesh,
        scratch_types=[
            pltpu.VMEM((BW,), jnp.int32),
            pltpu.VMEM((BW,), jnp.int32),
            pltpu.VMEM((BW * K,), jnp.int32),
            pltpu.VMEM((C * 8, D), jnp.float32),
            pltpu.VMEM((C, W), jnp.float32),
            pltpu.VMEM((CK, W), jnp.float32),
            pltpu.VMEM((C * 8, D), jnp.float32),
            pltpu.VMEM((C, W), jnp.float32),
            pltpu.VMEM((CK, W), jnp.float32),
            pltpu.VMEM((C * 2 * L,), jnp.float32),
            pltpu.SemaphoreType.DMA,
            pltpu.SemaphoreType.DMA,
        ],
        compiler_params=pltpu.CompilerParams(needs_layout_passes=False),
    )
    def sg_kernel(tw_hbm, cw_hbm, neg_hbm, tt_hbm, ct_hbm, out_hbm,
                  idx_t, idx_c, idx_n, rows_t_a, rows_c_a, rows_n_a,
                  rows_t_b, rows_c_b, rows_n_b, acc, sem_a, sem_b):
        wid = lax.axis_index("s") * NC + lax.axis_index("c")
        base_w = wid * BW

        pltpu.sync_copy(tw_hbm.at[pl.ds(base_w, BW)], idx_t)
        pltpu.sync_copy(cw_hbm.at[pl.ds(base_w, BW)], idx_c)
        pltpu.sync_copy(neg_hbm.at[pl.ds(base_w * K, BW * K)], idx_n)

        lane = lax.iota(jnp.int32, L)

        def issue(ci, rows_t, rows_c, rows_n, sem):
            base = ci * C
            tvec = idx_t[pl.ds(base, L)]
            ta = (tvec >> 3) << 3
            pltpu.async_copy(ct_hbm.at[idx_c.at[pl.ds(base, C)]], rows_c, sem)
            for e in range(C):
                pltpu.async_copy(
                    tt_hbm.at[pl.ds(pl.multiple_of(ta[e], 8), 8)],
                    rows_t.at[pl.ds(e * 8, 8)], sem)
            for j in range(NSTR):
                n = min(128, CK - j * 128)
                pltpu.async_copy(
                    ct_hbm.at[idx_n.at[pl.ds(base * K + j * 128, n)]],
                    rows_n.at[pl.ds(j * 128, n)], sem)

        def drain(rows_t, rows_c, rows_n, sem):
            pltpu.make_async_copy(tt_hbm.at[pl.ds(0, C * 8)], rows_t, sem).wait()
            pltpu.make_async_copy(ct_hbm.at[pl.ds(0, C)], rows_c, sem).wait()
            pltpu.make_async_copy(ct_hbm.at[pl.ds(0, CK)], rows_n, sem).wait()

        def compute(ci, rows_t, rows_c, rows_n):
            base = ci * C
            tvec = idx_t[pl.ds(base, L)]
            for e in range(C):
                st = tvec[e] & 7
                t = [rows_t[e * 8 + st, pl.ds(q * L, L)] for q in range(Q)]
                c = [rows_c[e, pl.ds(q * L, L)] for q in range(Q)]
                p = t[0] * c[0]
                for q in range(1, Q):
                    p = p + t[q] * c[q]
                v0 = jnp.where(lane == 0, jnp.sum(p), 0.0)
                v1 = jnp.zeros((L,), jnp.float32)
                for k in range(K):
                    j = e * K + k
                    s = t[0] * rows_n[j, pl.ds(0, L)]
                    for q in range(1, Q):
                        s = s + t[q] * rows_n[j, pl.ds(q * L, L)]
                    col = 1 + k
                    if col < L:
                        v0 = jnp.where(lane == col, jnp.sum(s), v0)
                    else:
                        v1 = jnp.where(lane == col - L, jnp.sum(s), v1)
                acc[pl.ds(e * 2 * L, L)] = v0
                acc[pl.ds(e * 2 * L + L, L)] = v1
            pltpu.sync_copy(
                acc, out_hbm.at[pl.ds((base_w + base) * 2 * L, C * 2 * L)])

        bufs_a = (rows_t_a, rows_c_a, rows_n_a)
        bufs_b = (rows_t_b, rows_c_b, rows_n_b)

        issue(0, *bufs_a, sem_a)

        def pair_body(h, carry):
            ca = 2 * h
            issue(ca + 1, *bufs_b, sem_b)
            drain(*bufs_a, sem_a)
            compute(ca, *bufs_a)
            issue(jnp.where(ca + 2 >= NCH, 0, ca + 2), *bufs_a, sem_a)
            drain(*bufs_b, sem_b)
            compute(ca + 1, *bufs_b)
            return carry

        lax.fori_loop(0, NCH // 2, pair_body, 0)
        drain(*bufs_a, sem_a)

    return sg_kernel


def kernel(target_word, context_word, negative_samples, target_table, context_table):
    B = target_word.shape[0]
    K = negative_samples.shape[1]
    V, D = target_table.shape
    tw = target_word.astype(jnp.int32)
    cw = context_word.astype(jnp.int32)
    neg = negative_samples.astype(jnp.int32).reshape(B * K)
    ctp = jnp.pad(context_table, ((0, 0), (0, W - D)))
    sg = _make_sc_kernel(B, K, D, V)
    out = sg(tw, cw, neg, target_table, ctp)
    return out.reshape(B, 2 * L)[:, :1 + K]
